# baseline (speedup 1.0000x reference)
import functools

import jax
import jax.numpy as jnp
from jax import lax
from jax.experimental import pallas as pl
from jax.experimental.pallas import tpu as pltpu
from jax.experimental.pallas import tpu_sc as plsc


def _phase_a(lt, wt, K=16384):
    D, V = lt.shape

    def body(lt_ref, wt_ref, t2_ref):
        ltb = lt_ref[...]
        wtb = wt_ref[...]
        e = jnp.exp(jnp.minimum(wtb, -wtb))
        sp = jnp.maximum(-wtb, 0.0) + jnp.log1p(e)
        negb = ltb - sp
        t2_ref[...] = jnp.concatenate(
            [ltb[:, :K // 2].T, ltb[:, K // 2:].T,
             negb[:, :K // 2].T, negb[:, K // 2:].T], axis=1)

    grid = pl.cdiv(V, K)
    return pl.pallas_call(
        body,
        grid=(grid,),
        in_specs=[pl.BlockSpec((D, K), lambda i: (0, i)),
                  pl.BlockSpec((D, K), lambda i: (0, i))],
        out_specs=pl.BlockSpec((K // 2, 4 * D), lambda i: (i, 0)),
        out_shape=jax.ShapeDtypeStruct((grid * (K // 2), 4 * D),
                                       jnp.float32),
    )(lt, wt)


def _make_phase_b(N, L, B, NC, NS, AK=16384):
    SH = AK.bit_length() - 1
    HM = AK // 2 - 1
    NW = NC * NS
    U = N // 128
    upw = U // NW
    NBUF = 4
    assert upw % NBUF == 0
    mesh = plsc.VectorSubcoreMesh(core_axis_name="c", subcore_axis_name="s")

    @functools.partial(
        pl.kernel,
        out_type=jax.ShapeDtypeStruct((2 * L, 32, B), jnp.float32),
        mesh=mesh,
        scratch_types=[
            pltpu.VMEM((N // NW,), jnp.int32),
            pltpu.VMEM((NBUF * 256,), jnp.int32),
            pltpu.VMEM((NBUF * 128, 128), jnp.float32),
            pltpu.VMEM((2 * 2, 32, 128), jnp.float32),
            pltpu.SemaphoreType.DMA((NBUF,)),
            pltpu.SemaphoreType.DMA((4,)),
        ],
        compiler_params=pltpu.CompilerParams(needs_layout_passes=False),
    )
    def gather_t(x_hbm, t2_hbm, out_hbm, iv, wv, gv, sv, sem_g, sem_o):
        wid = lax.axis_index("s") * NC + lax.axis_index("c")
        u0 = wid * upw
        lane = lax.iota(jnp.int32, 16)

        pltpu.sync_copy(x_hbm.at[pl.ds(u0 * 128, upw * 128)], iv)

        def prep_and_fire(u, b):
            r = (u - u0) * 128
            for j in range(8):
                v16 = iv[pl.ds(r + 16 * j, 16)]
                wv[pl.ds(b * 256 + 16 * j, 16)] = (
                    (v16 >> SH) * (AK // 2) + (v16 & HM))
                wv[pl.ds(b * 256 + 128 + 16 * j, 16)] = (
                    (v16 >> (SH - 1)) & 1) * 32
            return pltpu.async_copy(
                t2_hbm.at[wv.at[pl.ds(b * 256, 128)]],
                gv.at[pl.ds(b * 128, 128)], sem_g.at[b])

        def unit_body(u, b, b2, first):
            pltpu.make_async_copy(
                t2_hbm.at[wv.at[pl.ds(b * 256, 128)]],
                gv.at[pl.ds(b * 128, 128)], sem_g.at[b]).wait()
            un = jnp.minimum(u + 2, u0 + upw - 1)
            prep_and_fire(un, (b + 2) % NBUF)
            if not first:
                for h in range(2):
                    pltpu.make_async_copy(
                        sv.at[b2 * 2 + h],
                        out_hbm.at[0, :, pl.ds(0, 128)],
                        sem_o.at[b2 * 2 + h]).wait()
            ul = u // 128
            ub = u % 128
            for h in range(2):
                for j in range(8):
                    row16 = lane + (b * 128 + 16 * j)
                    col_base = wv[pl.ds(b * 256 + 128 + 16 * j, 16)]

                    @plsc.parallel_loop(0, 32, unroll=8)
                    def d_body(d):
                        vec = plsc.load_gather(
                            gv, [row16, col_base + (h * 64 + d)])
                        sv[b2 * 2 + h, d, pl.ds(16 * j, 16)] = vec
                pltpu.async_copy(
                    sv.at[b2 * 2 + h],
                    out_hbm.at[ul * 2 + h, :, pl.ds(ub * 128, 128)],
                    sem_o.at[b2 * 2 + h])

        prep_and_fire(u0, 0)
        prep_and_fire(u0 + 1, 1)

        for k in range(NBUF):
            unit_body(u0 + k, k % NBUF, k % 2, first=(k < 2))

        def loop_body(i, carry):
            ub0 = u0 + NBUF + i * NBUF
            for k in range(NBUF):
                unit_body(ub0 + k, k % NBUF, (NBUF + k) % 2, first=False)
            return carry

        lax.fori_loop(0, upw // NBUF - 1, loop_body, 0)

        for b in range(2):
            pltpu.make_async_copy(
                t2_hbm.at[wv.at[pl.ds(b * 256, 128)]],
                gv.at[pl.ds(b * 128, 128)], sem_g.at[b]).wait()
        for s in range(4):
            pltpu.make_async_copy(
                sv.at[s],
                out_hbm.at[0, :, pl.ds(0, 128)],
                sem_o.at[s]).wait()

    return gather_t


def kernel(x, logits, weight):
    B, L = x.shape
    V, D = logits.shape
    N = B * L
    info = plsc.get_sparse_core_info()
    NC, NS = info.num_cores, info.num_subcores
    t2 = _phase_a(logits.T, weight.T)
    xf = x.T.reshape(N).astype(jnp.int32)
    g = _make_phase_b(N, L, B, NC, NS)(xf, t2)
    return g.reshape(L, 2, D, B).transpose(3, 0, 1, 2)


# speedup vs baseline: 1.0155x; 1.0155x over previous
import functools

import jax
import jax.numpy as jnp
from jax import lax
from jax.experimental import pallas as pl
from jax.experimental.pallas import tpu as pltpu
from jax.experimental.pallas import tpu_sc as plsc


def _phase_a(lt, wt, K=16384):
    D, V = lt.shape

    def body(lt_ref, wt_ref, t2_ref):
        ltb = lt_ref[...]
        wtb = wt_ref[...]
        e = jnp.exp(jnp.minimum(wtb, -wtb))
        sp = jnp.maximum(-wtb, 0.0) + jnp.log1p(e)
        negb = ltb - sp
        t2_ref[...] = jnp.concatenate(
            [ltb[:, :K // 2].T, ltb[:, K // 2:].T,
             negb[:, :K // 2].T, negb[:, K // 2:].T], axis=1)

    grid = pl.cdiv(V, K)
    return pl.pallas_call(
        body,
        grid=(grid,),
        in_specs=[pl.BlockSpec((D, K), lambda i: (0, i)),
                  pl.BlockSpec((D, K), lambda i: (0, i))],
        out_specs=pl.BlockSpec((K // 2, 4 * D), lambda i: (i, 0)),
        out_shape=jax.ShapeDtypeStruct((grid * (K // 2), 4 * D),
                                       jnp.float32),
    )(lt, wt)


def _make_phase_b(N, L, B, NC, NS, AK=16384):
    SH = AK.bit_length() - 1
    HM = AK // 2 - 1
    NW = NC * NS
    U = N // 128
    upw = U // NW
    NBUF = 4
    assert upw % NBUF == 0
    mesh = plsc.VectorSubcoreMesh(core_axis_name="c", subcore_axis_name="s")

    @functools.partial(
        pl.kernel,
        out_type=jax.ShapeDtypeStruct((2 * L, 32, B), jnp.float32),
        mesh=mesh,
        scratch_types=[
            pltpu.VMEM((N // NW,), jnp.int32),
            pltpu.VMEM((NBUF * 256,), jnp.int32),
            pltpu.VMEM((NBUF * 128, 128), jnp.float32),
            pltpu.VMEM((2 * 2, 32, 128), jnp.float32),
            pltpu.SemaphoreType.DMA((NBUF,)),
            pltpu.SemaphoreType.DMA((2,)),
        ],
        compiler_params=pltpu.CompilerParams(needs_layout_passes=False),
    )
    def gather_t(x_hbm, t2_hbm, out_hbm, iv, wv, gv, sv, sem_g, sem_o):
        wid = lax.axis_index("s") * NC + lax.axis_index("c")
        u0 = wid * upw
        lane = lax.iota(jnp.int32, 16)

        pltpu.sync_copy(x_hbm.at[pl.ds(u0 * 128, upw * 128)], iv)

        def prep_and_fire(u, b):
            r = (u - u0) * 128
            for j in range(8):
                v16 = iv[pl.ds(r + 16 * j, 16)]
                wv[pl.ds(b * 256 + 16 * j, 16)] = (
                    (v16 >> SH) * (AK // 2) + (v16 & HM))
                wv[pl.ds(b * 256 + 128 + 16 * j, 16)] = (
                    (v16 >> (SH - 1)) & 1) * 32
            return pltpu.async_copy(
                t2_hbm.at[wv.at[pl.ds(b * 256, 128)]],
                gv.at[pl.ds(b * 128, 128)], sem_g.at[b])

        def unit_body(u, b, b2, first):
            pltpu.make_async_copy(
                t2_hbm.at[wv.at[pl.ds(b * 256, 128)]],
                gv.at[pl.ds(b * 128, 128)], sem_g.at[b]).wait()
            un = jnp.minimum(u + 2, u0 + upw - 1)
            prep_and_fire(un, (b + 2) % NBUF)
            if not first:
                pltpu.make_async_copy(
                    sv.at[pl.ds(b2 * 2, 2)],
                    out_hbm.at[pl.ds(0, 2), :, pl.ds(0, 128)],
                    sem_o.at[b2]).wait()
            for j in range(8):
                row16 = lane + (b * 128 + 16 * j)
                col_base = wv[pl.ds(b * 256 + 128 + 16 * j, 16)]

                @plsc.parallel_loop(0, 32, unroll=8)
                def d_body(d):
                    for h in range(2):
                        vec = plsc.load_gather(
                            gv, [row16, col_base + (h * 64 + d)])
                        sv[b2 * 2 + h, d, pl.ds(16 * j, 16)] = vec
            ul = u // 128
            ub = u % 128
            return pltpu.async_copy(
                sv.at[pl.ds(b2 * 2, 2)],
                out_hbm.at[pl.ds(ul * 2, 2), :, pl.ds(ub * 128, 128)],
                sem_o.at[b2])

        prep_and_fire(u0, 0)
        prep_and_fire(u0 + 1, 1)

        for k in range(NBUF):
            unit_body(u0 + k, k % NBUF, k % 2, first=(k < 2))

        def loop_body(i, carry):
            ub0 = u0 + NBUF + i * NBUF
            for k in range(NBUF):
                unit_body(ub0 + k, k % NBUF, (NBUF + k) % 2, first=False)
            return carry

        lax.fori_loop(0, upw // NBUF - 1, loop_body, 0)

        for b in range(2):
            pltpu.make_async_copy(
                t2_hbm.at[wv.at[pl.ds(b * 256, 128)]],
                gv.at[pl.ds(b * 128, 128)], sem_g.at[b]).wait()
        for b2 in range(2):
            pltpu.make_async_copy(
                sv.at[pl.ds(b2 * 2, 2)],
                out_hbm.at[pl.ds(0, 2), :, pl.ds(0, 128)],
                sem_o.at[b2]).wait()

    return gather_t


def kernel(x, logits, weight):
    B, L = x.shape
    V, D = logits.shape
    N = B * L
    info = plsc.get_sparse_core_info()
    NC, NS = info.num_cores, info.num_subcores
    t2 = _phase_a(logits.T, weight.T)
    xf = x.T.reshape(N).astype(jnp.int32)
    g = _make_phase_b(N, L, B, NC, NS)(xf, t2)
    return g.reshape(L, 2, D, B).transpose(3, 0, 1, 2)
